# baseline (speedup 1.0000x reference)
import functools

import jax
import jax.numpy as jnp
from jax import lax
from jax.experimental import pallas as pl
from jax.experimental.pallas import tpu as pltpu

NN = 10000
NG = 100
NPER = 100
NE = 160000
DIN = 128
HID = 256
NA = 128
GIN_L = 4

_INTERP = False


_RB = 2000
_NRB = NN // _RB


def _mlp_body(h_ref, msg_ref, deg_ref, w1_ref, b1_ref, w2_ref, b2_ref,
              h3_ref, sum_ref, ssq_ref):
    i = pl.program_id(0)
    deg = jnp.maximum(deg_ref[...], 1.0)
    hin = h_ref[...] + msg_ref[...] / deg
    t = jnp.maximum(
        jnp.dot(hin, w1_ref[...], preferred_element_type=jnp.float32,
                precision=lax.Precision.HIGHEST) + b1_ref[...], 0.0)
    h2 = jnp.dot(t, w2_ref[...], preferred_element_type=jnp.float32,
                 precision=lax.Precision.HIGHEST) + b2_ref[...]
    h3 = jnp.maximum(h2, 0.0)
    h3_ref[...] = h3

    @pl.when(i == 0)
    def _():
        sum_ref[...] = jnp.zeros_like(sum_ref)
        ssq_ref[...] = jnp.zeros_like(ssq_ref)

    sum_ref[...] += jnp.sum(h3, axis=0, keepdims=True)
    ssq_ref[...] += jnp.sum(h3 * h3, axis=0, keepdims=True)


def _bn_body(h3_ref, sum_ref, ssq_ref, gam_ref, bet_ref, np_ref,
             h_out_ref, np_out_ref):
    mu = sum_ref[...] * (1.0 / NN)
    var = jnp.maximum(ssq_ref[...] * (1.0 / NN) - mu * mu, 0.0)
    h3 = h3_ref[...]
    hbn = gam_ref[...] * (h3 - mu) / jnp.sqrt(var + 1e-5) + bet_ref[...]
    h_out_ref[...] = hbn
    np_out_ref[...] = np_ref[...] + hbn


def _layer_call(h, msg, deg, W1, b1, W2, b2, gam, bet, node_pool):
    din = h.shape[1]
    row = lambda d: pl.BlockSpec((_RB, d), lambda i: (i, 0))
    full = lambda a: pl.BlockSpec(a.shape, lambda i: (0,) * a.ndim)
    acc = pl.BlockSpec((1, HID), lambda i: (0, 0))
    b1r, b2r = b1.reshape(1, -1), b2.reshape(1, -1)
    h3, s, sq = pl.pallas_call(
        _mlp_body,
        grid=(_NRB,),
        in_specs=[row(din), row(din), pl.BlockSpec((_RB, 1), lambda i: (i, 0)),
                  full(W1), full(b1r), full(W2), full(b2r)],
        out_specs=(row(HID), acc, acc),
        out_shape=(jax.ShapeDtypeStruct((NN, HID), jnp.float32),
                   jax.ShapeDtypeStruct((1, HID), jnp.float32),
                   jax.ShapeDtypeStruct((1, HID), jnp.float32)),
        interpret=_INTERP,
    )(h, msg, deg, W1, b1r, W2, b2r)
    gamr, betr = gam.reshape(1, -1), bet.reshape(1, -1)
    return pl.pallas_call(
        _bn_body,
        grid=(_NRB,),
        in_specs=[row(HID), full(s), full(sq), full(gamr), full(betr),
                  row(HID)],
        out_specs=(row(HID), row(HID)),
        out_shape=(jax.ShapeDtypeStruct((NN, HID), jnp.float32),
                   jax.ShapeDtypeStruct((NN, HID), jnp.float32)),
        interpret=_INTERP,
    )(h3, s, sq, gamr, betr, node_pool)


def _embed_body(np_ref, ge_ref):
    node_g = lax.broadcasted_iota(jnp.int32, (NG, NN), 1) // NPER
    gid = lax.broadcasted_iota(jnp.int32, (NG, NN), 0)
    P = (node_g == gid).astype(jnp.float32)
    ge_ref[...] = jnp.dot(P, np_ref[...], preferred_element_type=jnp.float32,
                          precision=lax.Precision.HIGHEST) * (1.0 / NPER)


def _embed_call(node_pool):
    return pl.pallas_call(
        _embed_body,
        out_shape=jax.ShapeDtypeStruct((NG, HID), jnp.float32),
        interpret=_INTERP,
    )(node_pool)


def _head_body(np_ref, ge_ref, fa0c_ref, fa1c_ref, fa0r_ref, fa1r_ref, gum_ref,
               wa0_ref, wa1_ref, wa2_ref, ba_ref, wb_ref, bb_ref,
               wa2p_ref, ba2_ref, wb2_ref, bb2_ref,
               wa3_ref, ba3_ref, wb3t_ref, bb3_ref,
               act_ref, lp_ref, ent_ref):
    g = pl.program_id(0)
    hp = lax.Precision.HIGHEST
    npg = np_ref[0]
    lane_iota_n = lax.broadcasted_iota(jnp.int32, (NA, NPER), 1)
    oh0 = (fa0c_ref[0] == lane_iota_n).astype(jnp.float32)
    oh1 = (fa1c_ref[0] == lane_iota_n).astype(jnp.float32)
    e0 = jnp.dot(oh0, npg, preferred_element_type=jnp.float32, precision=hp)
    e1 = jnp.dot(oh1, npg, preferred_element_type=jnp.float32, precision=hp)
    ge = ge_ref[0]
    u = (jnp.dot(e0, wa0_ref[...], preferred_element_type=jnp.float32, precision=hp)
         + jnp.dot(e1, wa1_ref[...], preferred_element_type=jnp.float32, precision=hp)
         + jnp.dot(ge, wa2_ref[...], preferred_element_type=jnp.float32, precision=hp)
         + ba_ref[...])
    u = jnp.maximum(u, 0.0)
    v = jnp.dot(u, wb_ref[...], preferred_element_type=jnp.float32, precision=hp) + bb_ref[...]
    u2 = jnp.maximum(jnp.dot(v, wa2p_ref[...], preferred_element_type=jnp.float32, precision=hp) + ba2_ref[...], 0.0)
    v2 = jnp.dot(u2, wb2_ref[...], preferred_element_type=jnp.float32, precision=hp) + bb2_ref[...]
    u3 = jnp.maximum(jnp.dot(v2, wa3_ref[...], preferred_element_type=jnp.float32, precision=hp) + ba3_ref[...], 0.0)
    scores = lax.dot_general(wb3t_ref[...], u3, (((1,), (1,)), ((), ())),
                             preferred_element_type=jnp.float32, precision=hp)
    scores = scores + bb3_ref[...]
    m = jnp.max(scores, axis=1, keepdims=True)
    ex = jnp.exp(scores - m)
    logits = scores - m - jnp.log(jnp.sum(ex, axis=1, keepdims=True))
    pi = jnp.exp(logits)
    z = scores + gum_ref[0]
    zmax = jnp.max(z, axis=1, keepdims=True)
    lane_iota = lax.broadcasted_iota(jnp.int32, (1, NA), 1)
    idx = jnp.min(jnp.where(z == zmax, lane_iota, NA), axis=1, keepdims=True)
    oh = (lane_iota == idx).astype(jnp.float32)
    s0 = jnp.sum(oh * fa0r_ref[0].astype(jnp.float32), axis=1, keepdims=True)
    s1 = jnp.sum(oh * fa1r_ref[0].astype(jnp.float32), axis=1, keepdims=True)
    act_ref[0] = jnp.concatenate([s0, s1], axis=1).astype(jnp.int32)
    lp_ref[0] = jnp.sum(oh * logits, axis=1, keepdims=True)
    ent = -jnp.sum(pi * logits, axis=1, keepdims=True)

    @pl.when(g == 0)
    def _():
        ent_ref[...] = jnp.zeros_like(ent_ref)

    ent_ref[...] += ent * (1.0 / NG)


def _head_call(node_pool, graph_embed, fa, gumbel, policy_params):
    (Wa, ba, Wb, bb), (Wa2, ba2, Wb2, bb2), (Wa3, ba3, Wb3, bb3) = policy_params
    fa0c = fa[:, :, 0:1]
    fa1c = fa[:, :, 1:2]
    fa0r = fa[:, :, 0].reshape(NG, 1, NA)
    fa1r = fa[:, :, 1].reshape(NG, 1, NA)
    ge3 = graph_embed.reshape(NG, 1, HID)
    gum3 = gumbel.reshape(NG, 1, NA)
    row3 = lambda i: pl.BlockSpec((1, 1, NA), lambda g: (g, 0, 0))
    full = lambda a: pl.BlockSpec(a.shape, lambda g: (0,) * a.ndim)
    wspecs = []
    wargs = []
    for w in (Wa[:HID], Wa[HID:2 * HID], Wa[2 * HID:], ba.reshape(1, -1),
              Wb, bb.reshape(1, -1), Wa2, ba2.reshape(1, -1), Wb2,
              bb2.reshape(1, -1), Wa3, ba3.reshape(1, -1),
              Wb3.reshape(1, -1), bb3.reshape(1, 1)):
        wargs.append(w)
        wspecs.append(full(w))
    return pl.pallas_call(
        _head_body,
        grid=(NG,),
        in_specs=[
            pl.BlockSpec((1, NPER, HID), lambda g: (g, 0, 0)),
            pl.BlockSpec((1, 1, HID), lambda g: (g, 0, 0)),
            pl.BlockSpec((1, NA, 1), lambda g: (g, 0, 0)),
            pl.BlockSpec((1, NA, 1), lambda g: (g, 0, 0)),
            row3(0), row3(1), row3(2),
        ] + wspecs,
        out_specs=(
            pl.BlockSpec((1, 1, 2), lambda g: (g, 0, 0)),
            pl.BlockSpec((1, 1, 1), lambda g: (g, 0, 0)),
            pl.BlockSpec((1, 1), lambda g: (0, 0)),
        ),
        out_shape=(
            jax.ShapeDtypeStruct((NG, 1, 2), jnp.int32),
            jax.ShapeDtypeStruct((NG, 1, 1), jnp.float32),
            jax.ShapeDtypeStruct((1, 1), jnp.float32),
        ),
        interpret=_INTERP,
    )(node_pool.reshape(NG, NPER, HID), ge3, fa0c, fa1c, fa0r, fa1r, gum3,
      *wargs)


def _segment_mean_msg(h, src, dst):
    return jax.ops.segment_sum(h[src], dst, num_segments=NN)


def _degree(src, dst):
    return jax.ops.segment_sum(jnp.ones((NE,), jnp.float32), dst,
                               num_segments=NN)


def kernel(x, edge_index, batch, feasible_actions, gin_params, bn_params,
           policy_params):
    src, dst = edge_index[0], edge_index[1]
    deg = _degree(src, dst).reshape(NN, 1)
    h = x
    node_pool = jnp.zeros((NN, HID), jnp.float32)
    for i in range(GIN_L):
        msg = _segment_mean_msg(h, src, dst)
        W1, b1, W2, b2 = gin_params[i]
        gam, bet = bn_params[i]
        h, node_pool = _layer_call(h, msg, deg, W1, b1, W2, b2, gam, bet,
                                   node_pool)
    graph_embed = _embed_call(node_pool)
    gumbel = jax.random.gumbel(jax.random.key(42), (NG, NA), jnp.float32)
    act3, lp3, ent = _head_call(node_pool, graph_embed, feasible_actions,
                                gumbel, policy_params)
    return (act3.reshape(NG, 2), lp3.reshape(NG, 1), ent.reshape(()))
